# Initial kernel scaffold; baseline (speedup 1.0000x reference)
import jax
import jax.numpy as jnp
from jax.experimental import pallas as pl


def kernel(x, edge_index, W1l, b1, W1r, gamma, beta, W2l, b2, W2r):
    raise NotImplementedError("write your pallas kernel here")


# speedup vs baseline: 9.0676x; 9.0676x over previous
import functools

import jax
import jax.numpy as jnp
from jax import lax
from jax.experimental import pallas as pl
from jax.experimental.pallas import tpu as pltpu
from jax.experimental.pallas import tpu_sc as plsc

_N = 10000
_E = 320000
_DIN = 128
_H = 64

_NC = 2
_NS = 16
_L = 16
_NW = _NC * _NS
_NP = 10240
_SL = _NP // _NS
_EPT = _E // _NW
_K = 80
_NCH = _EPT // _K

_mesh = plsc.VectorSubcoreMesh(core_axis_name="core", subcore_axis_name="subcore")


@functools.partial(
    pl.kernel,
    out_type=jax.ShapeDtypeStruct((_NC, _NP), jnp.float32),
    mesh=_mesh,
    scratch_types=[
        pltpu.VMEM_SHARED((_NP,), jnp.float32),
        pltpu.VMEM((1, _K), jnp.int32),
        pltpu.VMEM((_K,), jnp.float32),
    ],
)
def _sc_degree(dst_hbm, zero1_hbm, cnt_hbm, cnt_sh, idx_v, ones_v):
    cid = lax.axis_index("core")
    sid = lax.axis_index("subcore")
    wid = cid * _NS + sid
    row = sid * _SL
    pltpu.sync_copy(zero1_hbm.at[pl.ds(row, _SL)], cnt_sh.at[pl.ds(row, _SL)])

    @pl.loop(0, _K, step=_L)
    def _(j):
        ones_v[pl.ds(j, _L)] = jnp.ones((_L,), jnp.float32)

    plsc.subcore_barrier()
    base0 = wid * _EPT

    @pl.loop(0, _NCH)
    def _(i):
        pltpu.sync_copy(dst_hbm.at[pl.ds(base0 + i * _K, _K)], idx_v.at[0])
        pltpu.sync_copy(ones_v, cnt_sh.at[idx_v.at[0]], add=True)

    plsc.subcore_barrier()
    pltpu.sync_copy(cnt_sh.at[pl.ds(row, _SL)], cnt_hbm.at[cid, pl.ds(row, _SL)])


@functools.partial(
    pl.kernel,
    out_type=(
        jax.ShapeDtypeStruct((_NC, _NP, _H), jnp.float32),
        jax.ShapeDtypeStruct((_NC, _NP), jnp.float32),
        jax.ShapeDtypeStruct((_NC, _NP), jnp.float32),
    ),
    mesh=_mesh,
    compiler_params=pltpu.CompilerParams(
        needs_layout_passes=False, use_tc_tiling_on_sc=False),
    scratch_types=[
        pltpu.VMEM_SHARED((_NP, _H), jnp.float32),
        pltpu.VMEM_SHARED((_NP,), jnp.float32),
        pltpu.VMEM_SHARED((_NP,), jnp.float32),
        pltpu.VMEM((_NP,), jnp.float32),
        pltpu.VMEM((1, _K), jnp.int32),
        pltpu.VMEM((1, _K), jnp.int32),
        pltpu.VMEM((_K, _H), jnp.float32),
        pltpu.VMEM((_K,), jnp.float32),
        pltpu.VMEM((_SL,), jnp.float32),
        pltpu.VMEM((_SL,), jnp.float32),
        pltpu.VMEM((_SL,), jnp.float32),
    ],
)
def _sc_aggregate(src_hbm, dst_hbm, y1_hbm, cntp_hbm, zero2_hbm, zero1_hbm,
                  seg_hbm, c_hbm, inv_hbm,
                  seg_sh, c_sh, inv_sh, inv_v, src_v, dst_v, rows_v, vals_v,
                  cnt0_v, cnt1_v, invs_v):
    cid = lax.axis_index("core")
    sid = lax.axis_index("subcore")
    wid = cid * _NS + sid
    row = sid * _SL

    pltpu.sync_copy(zero2_hbm.at[pl.ds(row, _SL)], seg_sh.at[pl.ds(row, _SL)])
    pltpu.sync_copy(zero1_hbm.at[pl.ds(row, _SL)], c_sh.at[pl.ds(row, _SL)])

    pltpu.sync_copy(cntp_hbm.at[0, pl.ds(row, _SL)], cnt0_v)
    pltpu.sync_copy(cntp_hbm.at[1, pl.ds(row, _SL)], cnt1_v)

    @pl.loop(0, _SL, step=_L)
    def _(i):
        a = cnt0_v[pl.ds(i, _L)] + cnt1_v[pl.ds(i, _L)]
        invs_v[pl.ds(i, _L)] = 1.0 / jnp.maximum(a, 1.0)

    pltpu.sync_copy(invs_v, inv_sh.at[pl.ds(row, _SL)])
    pltpu.sync_copy(invs_v, inv_hbm.at[cid, pl.ds(row, _SL)])
    plsc.subcore_barrier()

    pltpu.sync_copy(inv_sh, inv_v)
    base0 = wid * _EPT

    @pl.loop(0, _NCH)
    def _(i):
        b = base0 + i * _K
        pltpu.sync_copy(src_hbm.at[pl.ds(b, _K)], src_v.at[0])
        pltpu.sync_copy(dst_hbm.at[pl.ds(b, _K)], dst_v.at[0])
        pltpu.sync_copy(y1_hbm.at[src_v.at[0]], rows_v)
        pltpu.sync_copy(rows_v, seg_sh.at[dst_v.at[0]], add=True)
        for j in range(_K // _L):
            iv = dst_v.at[0][pl.ds(j * _L, _L)]
            vals_v[pl.ds(j * _L, _L)] = plsc.load_gather(inv_v, [iv])
        pltpu.sync_copy(vals_v, c_sh.at[src_v.at[0]], add=True)

    plsc.subcore_barrier()
    pltpu.sync_copy(seg_sh.at[pl.ds(row, _SL)], seg_hbm.at[cid, pl.ds(row, _SL)])
    pltpu.sync_copy(c_sh.at[pl.ds(row, _SL)], c_hbm.at[cid, pl.ds(row, _SL)])


def _tc_pre_body(x_ref, w_ref, y1_ref, r1_ref):
    y = jnp.dot(x_ref[...], w_ref[...], preferred_element_type=jnp.float32)
    y1_ref[...] = y[:, :_H]
    r1_ref[...] = y[:, _H:]


_tc_pre = pl.pallas_call(
    _tc_pre_body,
    out_shape=(
        jax.ShapeDtypeStruct((_N, _H), jnp.float32),
        jax.ShapeDtypeStruct((_N, _H), jnp.float32),
    ),
)


def _tc_post_body(seg0_ref, seg1_ref, r1_ref, inv_ref, c0_ref, c1_ref,
                  b1_ref, g_ref, bt_ref, w2l_ref, w2r_ref, b2_ref, out_ref):
    z = ((seg0_ref[...] + seg1_ref[...]) * inv_ref[...]
         + r1_ref[...] + b1_ref[...])
    mean = jnp.mean(z, axis=0, keepdims=True)
    zc = z - mean
    var = jnp.mean(zc * zc, axis=0, keepdims=True)
    h = g_ref[...] * zc * lax.rsqrt(var + 1e-5) + bt_ref[...]
    h = jnp.maximum(h, 0.0)
    cc = c0_ref[...] + c1_ref[...]
    s1 = jnp.sum(h * cc, axis=0, keepdims=True) * (1.0 / _N)
    s2 = jnp.sum(h, axis=0, keepdims=True) * (1.0 / _N)
    out_ref[...] = (
        jnp.dot(s1, w2l_ref[...], preferred_element_type=jnp.float32)
        + jnp.dot(s2, w2r_ref[...], preferred_element_type=jnp.float32)
        + b2_ref[...]
    )


_tc_post = pl.pallas_call(
    _tc_post_body,
    out_shape=jax.ShapeDtypeStruct((1, _H), jnp.float32),
)


def kernel(x, edge_index, W1l, b1, W1r, gamma, beta, W2l, b2, W2r):
    src = edge_index[0]
    dst = edge_index[1]
    wcat = jnp.concatenate([W1l, W1r], axis=1)
    zero1 = jnp.zeros((_NP,), jnp.float32)
    zero2 = jnp.zeros((_NP, _H), jnp.float32)

    y1, r1 = _tc_pre(x, wcat)
    cntp = _sc_degree(dst, zero1)
    segp, cp, invp = _sc_aggregate(src, dst, y1, cntp, zero2, zero1)

    return _tc_post(
        segp[0, :_N], segp[1, :_N], r1,
        invp[0, :_N, None], cp[0, :_N, None], cp[1, :_N, None],
        b1[None, :], gamma[None, :], beta[None, :],
        W2l, W2r, b2[None, :],
    )


# speedup vs baseline: 26.8871x; 2.9652x over previous
import functools

import jax
import jax.numpy as jnp
from jax import lax
from jax.experimental import pallas as pl
from jax.experimental.pallas import tpu as pltpu
from jax.experimental.pallas import tpu_sc as plsc

_N = 10000
_E = 320000
_DIN = 128
_H = 64

_NC = 2
_NS = 16
_L = 16
_NW = _NC * _NS
_NP = 10240
_SL = _NP // _NS
_EPT = _E // _NW
_K = 80
_NCH = _EPT // _K
_ER = _E // _K
_NB = 4

_mesh = plsc.VectorSubcoreMesh(core_axis_name="core", subcore_axis_name="subcore")


@functools.partial(
    pl.kernel,
    out_type=jax.ShapeDtypeStruct((_NC, _NP), jnp.float32),
    mesh=_mesh,
    compiler_params=pltpu.CompilerParams(use_tc_tiling_on_sc=False),
    scratch_types=[
        pltpu.VMEM_SHARED((_NP,), jnp.float32),
        pltpu.VMEM((_NCH, _K), jnp.int32),
        pltpu.VMEM((_K,), jnp.float32),
        pltpu.SemaphoreType.DMA,
    ],
)
def _sc_degree(dst2_hbm, zero1_hbm, cnt_hbm, cnt_sh, idx_v, ones_v, sem):
    cid = lax.axis_index("core")
    sid = lax.axis_index("subcore")
    wid = cid * _NS + sid
    row = sid * _SL
    pltpu.sync_copy(zero1_hbm.at[pl.ds(row, _SL)], cnt_sh.at[pl.ds(row, _SL)])
    pltpu.sync_copy(dst2_hbm.at[pl.ds(wid * _NCH, _NCH)], idx_v)

    @pl.loop(0, _K, step=_L)
    def _(j):
        ones_v[pl.ds(j, _L)] = jnp.ones((_L,), jnp.float32)

    plsc.subcore_barrier()

    @pl.loop(0, _NCH, step=5)
    def _(i):
        for k in range(5):
            pltpu.async_copy(ones_v, cnt_sh.at[idx_v.at[i + k]], sem, add=True)
        for k in range(5):
            pltpu.make_async_copy(
                ones_v, cnt_sh.at[idx_v.at[i + k]], sem).wait()

    plsc.subcore_barrier()
    pltpu.sync_copy(cnt_sh.at[pl.ds(row, _SL)], cnt_hbm.at[cid, pl.ds(row, _SL)])


@functools.partial(
    pl.kernel,
    out_type=(
        jax.ShapeDtypeStruct((_NC, _NP, _H), jnp.float32),
        jax.ShapeDtypeStruct((_NC, _NP), jnp.float32),
    ),
    mesh=_mesh,
    compiler_params=pltpu.CompilerParams(
        needs_layout_passes=False, use_tc_tiling_on_sc=False),
    scratch_types=[
        pltpu.VMEM_SHARED((_NP, _H), jnp.float32),
        pltpu.VMEM_SHARED((_NP,), jnp.float32),
        pltpu.VMEM_SHARED((_NP,), jnp.float32),
        pltpu.VMEM((_NP,), jnp.float32),
        pltpu.VMEM((_NCH, _K), jnp.int32),
        pltpu.VMEM((_NCH, _K), jnp.int32),
        pltpu.VMEM((_NB, _K, _H), jnp.float32),
        pltpu.VMEM((_K,), jnp.float32),
        pltpu.VMEM((_SL,), jnp.float32),
        pltpu.VMEM((_SL,), jnp.float32),
        pltpu.VMEM((_SL,), jnp.float32),
        pltpu.VMEM((128, _H), jnp.float32),
        pltpu.SemaphoreType.DMA((_NB,)),
    ],
)
def _sc_aggregate(src2_hbm, dst2_hbm, y1_hbm, cntp_hbm, zero2_hbm, zero1_hbm,
                  seg_hbm, c_hbm,
                  seg_sh, c_sh, inv_sh, inv_v, src_v, dst_v, rows_v, vals_v,
                  cnt0_v, cnt1_v, invs_v, segb_v, gsem):
    cid = lax.axis_index("core")
    sid = lax.axis_index("subcore")
    wid = cid * _NS + sid
    row = sid * _SL

    pltpu.sync_copy(zero2_hbm.at[pl.ds(row, _SL)], seg_sh.at[pl.ds(row, _SL)])
    pltpu.sync_copy(zero1_hbm.at[pl.ds(row, _SL)], c_sh.at[pl.ds(row, _SL)])

    pltpu.sync_copy(src2_hbm.at[pl.ds(wid * _NCH, _NCH)], src_v)
    pltpu.sync_copy(dst2_hbm.at[pl.ds(wid * _NCH, _NCH)], dst_v)

    pltpu.sync_copy(cntp_hbm.at[0, pl.ds(row, _SL)], cnt0_v)
    pltpu.sync_copy(cntp_hbm.at[1, pl.ds(row, _SL)], cnt1_v)

    @pl.loop(0, _SL, step=_L)
    def _(i):
        a = cnt0_v[pl.ds(i, _L)] + cnt1_v[pl.ds(i, _L)]
        invs_v[pl.ds(i, _L)] = 1.0 / jnp.maximum(a, 1.0)

    pltpu.sync_copy(invs_v, inv_sh.at[pl.ds(row, _SL)])
    plsc.subcore_barrier()

    pltpu.sync_copy(inv_sh, inv_v)

    def _gather(i, b):
        pltpu.make_async_copy(
            y1_hbm.at[src_v.at[i]], rows_v.at[b], gsem.at[b]).start()

    def _process(i, b):
        pltpu.make_async_copy(
            y1_hbm.at[src_v.at[i]], rows_v.at[b], gsem.at[b]).wait()
        pltpu.sync_copy(rows_v.at[b], seg_sh.at[dst_v.at[i]], add=True)
        for j in range(_K // _L):
            iv = dst_v.at[i][pl.ds(j * _L, _L)]
            vals_v[pl.ds(j * _L, _L)] = plsc.load_gather(inv_v, [iv])
        pltpu.sync_copy(vals_v, c_sh.at[src_v.at[i]], add=True)

    for b in range(_NB):
        _gather(b, b)

    @pl.loop(0, _NCH - 1, step=_NB)
    def _(g):
        for b in range(_NB):
            i = g + b
            _process(i, b)

            @pl.when(i + _NB < _NCH)
            def _():
                _gather(i + _NB, b)

    _process(_NCH - 1, 0)

    plsc.subcore_barrier()

    @pl.loop(0, _SL, step=128)
    def _(r0):
        pltpu.sync_copy(seg_sh.at[pl.ds(row + r0, 128)], segb_v)

        @pl.loop(0, 128)
        def _(r):
            s = plsc.load_gather(invs_v, [jnp.full((_L,), r0 + r, jnp.int32)])
            for q in range(_H // _L):
                segb_v[r, pl.ds(q * _L, _L)] = segb_v[r, pl.ds(q * _L, _L)] * s

        pltpu.sync_copy(segb_v, seg_hbm.at[cid, pl.ds(row + r0, 128)])
    pltpu.sync_copy(c_sh.at[pl.ds(row, _SL)], c_hbm.at[cid, pl.ds(row, _SL)])


def _tc_pre_body(x_ref, wl_ref, wr_ref, y1_ref, r1_ref):
    y1 = jnp.dot(x_ref[...], wl_ref[...], preferred_element_type=jnp.float32)
    r1 = jnp.dot(x_ref[...], wr_ref[...], preferred_element_type=jnp.float32)
    y1_ref[pl.ds(0, _N), :] = y1
    r1_ref[pl.ds(0, _N), :] = r1
    pad = jnp.zeros((_NP - _N, _H), jnp.float32)
    y1_ref[pl.ds(_N, _NP - _N), :] = pad
    r1_ref[pl.ds(_N, _NP - _N), :] = pad


_tc_pre = pl.pallas_call(
    _tc_pre_body,
    out_shape=(
        jax.ShapeDtypeStruct((_NP, _H), jnp.float32),
        jax.ShapeDtypeStruct((_NP, _H), jnp.float32),
    ),
)


def _tc_post_body(segp_ref, cp_ref, r1_ref,
                  b1_ref, g_ref, bt_ref, w2l_ref, w2r_ref, b2_ref, out_ref):
    mask = (lax.broadcasted_iota(jnp.int32, (_NP, 1), 0) < _N).astype(
        jnp.float32)
    z = segp_ref[0] + segp_ref[1] + r1_ref[...] + b1_ref[...]
    mean = jnp.sum(z * mask, axis=0, keepdims=True) * (1.0 / _N)
    zc = z - mean
    var = jnp.sum(zc * zc * mask, axis=0, keepdims=True) * (1.0 / _N)
    h = g_ref[...] * zc * lax.rsqrt(var + 1e-5) + bt_ref[...]
    hm = jnp.maximum(h, 0.0) * mask
    cc = cp_ref[0:1, :] + cp_ref[1:2, :]
    s1 = jnp.dot(cc, hm, preferred_element_type=jnp.float32) * (1.0 / _N)
    s2 = jnp.sum(hm, axis=0, keepdims=True) * (1.0 / _N)
    out_ref[...] = (
        jnp.dot(s1, w2l_ref[...], preferred_element_type=jnp.float32)
        + jnp.dot(s2, w2r_ref[...], preferred_element_type=jnp.float32)
        + b2_ref[...]
    )


_tc_post = pl.pallas_call(
    _tc_post_body,
    out_shape=jax.ShapeDtypeStruct((1, _H), jnp.float32),
)


def kernel(x, edge_index, W1l, b1, W1r, gamma, beta, W2l, b2, W2r):
    src2 = edge_index[0].reshape(_ER, _K)
    dst2 = edge_index[1].reshape(_ER, _K)
    zero1 = jnp.zeros((_NP,), jnp.float32)
    zero2 = jnp.zeros((_NP, _H), jnp.float32)

    y1, r1 = _tc_pre(x, W1l, W1r)
    cntp = _sc_degree(dst2, zero1)
    segp, cp = _sc_aggregate(src2, dst2, y1, cntp, zero2, zero1)

    return _tc_post(
        segp, cp, r1,
        b1[None, :], gamma[None, :], beta[None, :],
        W2l, W2r, b2[None, :],
    )
